# Initial kernel scaffold; baseline (speedup 1.0000x reference)
import jax
import jax.numpy as jnp
from jax.experimental import pallas as pl


def kernel(x, edge_index, epoch, W1, b1, W2, b2, temp):
    raise NotImplementedError("write your pallas kernel here")


# speedup vs baseline: 7.8295x; 7.8295x over previous
import functools

import jax
import jax.numpy as jnp
from jax import lax
from jax.experimental import pallas as pl
from jax.experimental.pallas import tpu as pltpu
from jax.experimental.pallas import tpu_sc as plsc

N = 10000
E = 320000
DF = 128
DH = 128
DC = 40
DP = 48
NPAD = 10240
KCH = 10

NS = 16
CHUNK = 128
NCHUNK = 157
EPAD = NS * NCHUNK * CHUNK
ROWS_T = NPAD // NS
DEGW = 8

_mesh = plsc.VectorSubcoreMesh(core_axis_name="c", subcore_axis_name="s")
_sc_params = pltpu.CompilerParams(use_tc_tiling_on_sc=False)


def _coeffs(temp):
    kp1 = temp.shape[0]
    kc = kp1 - 1
    j = jnp.arange(kp1, dtype=jnp.float32)
    xj = jnp.cos((kc - j + 0.5) * jnp.pi / (kc + 1))
    i = jnp.arange(kp1, dtype=jnp.float32)
    tmat = jnp.cos(i[:, None] * jnp.arccos(xj)[None, :])
    return (2.0 / (kc + 1)) * (tmat @ jax.nn.relu(temp))


def _zero_my_slice(zer_v, acc_sh, sid, width_rows):
    full, rem = divmod(ROWS_T, width_rows)
    for j in range(full):
        pltpu.sync_copy(zer_v, acc_sh.at[pl.ds(sid * ROWS_T + j * width_rows,
                                               width_rows)])
    if rem:
        pltpu.sync_copy(zer_v.at[pl.ds(0, rem)],
                        acc_sh.at[pl.ds(sid * ROWS_T + full * width_rows, rem)])


@functools.partial(
    pl.kernel,
    out_type=jax.ShapeDtypeStruct((NPAD, DEGW), jnp.float32),
    mesh=_mesh,
    scratch_types=[
        pltpu.VMEM((NCHUNK, CHUNK), jnp.int32),
        pltpu.VMEM((CHUNK, DEGW), jnp.float32),
        pltpu.VMEM((CHUNK, DEGW), jnp.float32),
        pltpu.VMEM_SHARED((NPAD, DEGW), jnp.float32),
    ],
    compiler_params=_sc_params,
)
def _sc_degree(dst_hbm, ones_hbm, zeros_hbm, deg_hbm,
               dst_v, ones_v, zer_v, acc_sh):
    cid = lax.axis_index("c")
    sid = lax.axis_index("s")

    @pl.when(cid == 0)
    def _():
        pltpu.sync_copy(dst_hbm.at[sid], dst_v)
        pltpu.sync_copy(ones_hbm, ones_v)
        pltpu.sync_copy(zeros_hbm, zer_v)
        _zero_my_slice(zer_v, acc_sh, sid, CHUNK)
        plsc.subcore_barrier()

        def body(i, carry):
            pltpu.sync_copy(ones_v, acc_sh.at[dst_v.at[i]], add=True)
            return carry

        lax.fori_loop(0, NCHUNK, body, 0)
        plsc.subcore_barrier()
        pltpu.sync_copy(acc_sh.at[pl.ds(sid * ROWS_T, ROWS_T)],
                        deg_hbm.at[pl.ds(sid * ROWS_T, ROWS_T)])


@functools.partial(
    pl.kernel,
    out_type=jax.ShapeDtypeStruct((NPAD, DP), jnp.float32),
    mesh=_mesh,
    scratch_types=[
        pltpu.VMEM((NCHUNK, CHUNK), jnp.int32),
        pltpu.VMEM((NCHUNK, CHUNK), jnp.int32),
        pltpu.VMEM((CHUNK, DP), jnp.float32),
        pltpu.VMEM((CHUNK, DP), jnp.float32),
        pltpu.VMEM_SHARED((NPAD, DP), jnp.float32),
    ],
    compiler_params=_sc_params,
)
def _sc_round(src_hbm, dst_hbm, u_hbm, zeros_hbm, s_hbm,
              src_v, dst_v, rows_v, zer_v, s_sh):
    cid = lax.axis_index("c")
    sid = lax.axis_index("s")

    @pl.when(cid == 0)
    def _():
        pltpu.sync_copy(src_hbm.at[sid], src_v)
        pltpu.sync_copy(dst_hbm.at[sid], dst_v)
        pltpu.sync_copy(zeros_hbm, zer_v)
        _zero_my_slice(zer_v, s_sh, sid, CHUNK)
        plsc.subcore_barrier()

        def body(i, carry):
            pltpu.sync_copy(u_hbm.at[src_v.at[i]], rows_v)
            pltpu.sync_copy(rows_v, s_sh.at[dst_v.at[i]], add=True)
            return carry

        lax.fori_loop(0, NCHUNK, body, 0)
        plsc.subcore_barrier()
        pltpu.sync_copy(s_sh.at[pl.ds(sid * ROWS_T, ROWS_T)],
                        s_hbm.at[pl.ds(sid * ROWS_T, ROWS_T)])


_BR = 1024
_GRID = NPAD // _BR


def _row_spec(width):
    return pl.BlockSpec((_BR, width), lambda i: (i, 0))


def _full_spec(shape):
    return pl.BlockSpec(shape, lambda i: (0,) * len(shape))


def _smem_spec():
    return pl.BlockSpec(memory_space=pltpu.SMEM)


def _mlp_body(x_ref, w1_ref, b1_ref, w2_ref, b2_ref, deg_ref,
              h_ref, dinv_ref, u0_ref):
    hb = jnp.maximum(
        jnp.dot(x_ref[...], w1_ref[...], preferred_element_type=jnp.float32)
        + b1_ref[...], 0.0)
    hb = (jnp.dot(hb, w2_ref[...], preferred_element_type=jnp.float32)
          + b2_ref[...])
    deg = deg_ref[...][:, 0:1]
    dinv = jnp.where(deg > 0, lax.rsqrt(jnp.maximum(deg, 1e-12)), 0.0)
    dinv48 = dinv * jnp.ones((1, DP), jnp.float32)
    h_ref[...] = hb
    dinv_ref[...] = dinv48
    u0_ref[...] = hb * dinv48


def _tc_mlp(x, W1, b1, W2p, b2p, deg8):
    out_t = [jax.ShapeDtypeStruct((NPAD, DP), jnp.float32)] * 3
    return pl.pallas_call(
        _mlp_body,
        grid=(_GRID,),
        in_specs=[_row_spec(DF), _full_spec((DF, DH)), _full_spec((1, DH)),
                  _full_spec((DH, DP)), _full_spec((1, DP)), _row_spec(DEGW)],
        out_specs=[_row_spec(DP)] * 3,
        out_shape=out_t,
    )(x, W1, b1.reshape(1, DH), W2p, b2p.reshape(1, DP), deg8)


def _round1_body(s_ref, h_ref, dinv_ref, c_ref, tx_ref, out_ref, u_ref):
    tx1 = -dinv_ref[...] * s_ref[...]
    tx_ref[...] = tx1
    out_ref[...] = c_ref[0] * h_ref[...] + c_ref[1] * tx1
    u_ref[...] = dinv_ref[...] * tx1


def _tc_round1(s, h, dinv48, c01):
    out_t = [jax.ShapeDtypeStruct((NPAD, DP), jnp.float32)] * 3
    return pl.pallas_call(
        _round1_body,
        grid=(_GRID,),
        in_specs=[_row_spec(DP), _row_spec(DP), _row_spec(DP), _smem_spec()],
        out_specs=[_row_spec(DP)] * 3,
        out_shape=out_t,
    )(s, h, dinv48, c01)


def _roundk_body(s_ref, tx0_ref, acc_ref, dinv_ref, c_ref,
                 tx_ref, out_ref, u_ref):
    tx2 = -2.0 * dinv_ref[...] * s_ref[...] - tx0_ref[...]
    tx_ref[...] = tx2
    out_ref[...] = acc_ref[...] + c_ref[0] * tx2
    u_ref[...] = dinv_ref[...] * tx2


def _tc_roundk(s, tx0, acc, dinv48, ck):
    out_t = [jax.ShapeDtypeStruct((NPAD, DP), jnp.float32)] * 3
    return pl.pallas_call(
        _roundk_body,
        grid=(_GRID,),
        in_specs=[_row_spec(DP), _row_spec(DP), _row_spec(DP), _row_spec(DP),
                  _smem_spec()],
        out_specs=[_row_spec(DP)] * 3,
        out_shape=out_t,
    )(s, tx0, acc, dinv48, ck)


def kernel(x, edge_index, epoch, W1, b1, W2, b2, temp):
    del epoch
    coe = _coeffs(temp)
    xp = jnp.pad(x, ((0, NPAD - N), (0, 0)))

    src = edge_index[0].astype(jnp.int32)
    dst = edge_index[1].astype(jnp.int32)
    pad = EPAD - E
    src_t = jnp.concatenate([src, jnp.zeros((pad,), jnp.int32)]
                            ).reshape(NS, NCHUNK, CHUNK)
    dst_t = jnp.concatenate([dst, jnp.full((pad,), N, jnp.int32)]
                            ).reshape(NS, NCHUNK, CHUNK)

    ones_c = jnp.tile(
        jnp.eye(1, DEGW, dtype=jnp.float32), (CHUNK, 1))
    zeros8_c = jnp.zeros((CHUNK, DEGW), jnp.float32)
    zeros48_c = jnp.zeros((CHUNK, DP), jnp.float32)

    W2p = jnp.pad(W2, ((0, 0), (0, DP - DC)))
    b2p = jnp.pad(b2, (0, DP - DC))

    deg8 = _sc_degree(dst_t, ones_c, zeros8_c)
    h, dinv48, u = _tc_mlp(xp, W1, b1, W2p, b2p, deg8)

    s = _sc_round(src_t, dst_t, u, zeros48_c)
    c01 = jnp.stack([coe[0] / 2.0, coe[1]])
    tx1, out, u = _tc_round1(s, h, dinv48, c01)

    tx0 = h
    txprev = tx1
    for k in range(2, KCH + 1):
        s = _sc_round(src_t, dst_t, u, zeros48_c)
        txk, out, u = _tc_roundk(s, tx0, out, dinv48, coe[k].reshape(1))
        tx0, txprev = txprev, txk

    return out[:N, :DC]
